# Initial kernel scaffold; baseline (speedup 1.0000x reference)
import jax
import jax.numpy as jnp
from jax.experimental import pallas as pl


def kernel(x, pos, batch, edge_index, thresholds, W, b):
    raise NotImplementedError("write your pallas kernel here")


# speedup vs baseline: 2.8335x; 2.8335x over previous
import functools

import jax
import jax.numpy as jnp
from jax import lax
from jax.experimental import pallas as pl
from jax.experimental.pallas import tpu as pltpu
from jax.experimental.pallas import tpu_sc as plsc

NC = 2
NS = 16
NW = NC * NS
CHUNK = 80
GROUPS = CHUNK // 16
WIDTH = 136


def _nsqrt(a):
    i = lax.bitcast_convert_type(a, jnp.int32)
    i = 0x5F3759DF - lax.shift_right_logical(i, 1)
    y = lax.bitcast_convert_type(i, jnp.float32)
    for _ in range(4):
        y = y * (1.5 - 0.5 * a * y * y)
    return jnp.where(a > 0.0, a * y, 0.0)


def _make_mesh():
    return plsc.VectorSubcoreMesh(
        core_axis_name="c", subcore_axis_name="s", num_cores=NC, num_subcores=NS
    )


CHUNK_A = 400


def _sc_pos_dist(posx, posy, posz, edge_index, n_nodes, n_edges):
    epw = n_edges // NW
    nchunk = epw // CHUNK_A

    @functools.partial(
        pl.kernel,
        out_type=jax.ShapeDtypeStruct((n_edges,), jnp.float32),
        mesh=_make_mesh(),
        compiler_params=pltpu.CompilerParams(needs_layout_passes=False),
        scratch_types=[
            pltpu.VMEM((CHUNK_A,), jnp.int32),
            pltpu.VMEM((CHUNK_A,), jnp.int32),
            pltpu.VMEM((CHUNK_A,), jnp.float32),
            pltpu.VMEM((n_nodes,), jnp.float32),
            pltpu.VMEM((n_nodes,), jnp.float32),
            pltpu.VMEM((n_nodes,), jnp.float32),
        ],
    )
    def pos_kernel(px_hbm, py_hbm, pz_hbm, esrc_hbm, edst_hbm, out_hbm,
                   src_v, dst_v, dpo_v, px_v, py_v, pz_v):
        c = lax.axis_index("c")
        s = lax.axis_index("s")
        wid = s * NC + c
        pltpu.sync_copy(px_hbm, px_v)
        pltpu.sync_copy(py_hbm, py_v)
        pltpu.sync_copy(pz_hbm, pz_v)
        base = wid * epw

        def chunk_body(i, carry):
            off = base + i * CHUNK_A
            pltpu.sync_copy(esrc_hbm.at[pl.ds(off, CHUNK_A)], src_v)
            pltpu.sync_copy(edst_hbm.at[pl.ds(off, CHUNK_A)], dst_v)

            def group_body(g, carry2):
                src16 = src_v[pl.ds(g * 16, 16)]
                dst16 = dst_v[pl.ds(g * 16, 16)]
                dp2 = jnp.zeros((16,), jnp.float32)
                for pv in (px_v, py_v, pz_v):
                    a = plsc.load_gather(pv, [dst16])
                    b2 = plsc.load_gather(pv, [src16])
                    df = a - b2
                    dp2 = dp2 + df * df
                dpo_v[pl.ds(g * 16, 16)] = _nsqrt(dp2)
                return carry2

            lax.fori_loop(0, CHUNK_A // 16, group_body, 0)
            pltpu.sync_copy(dpo_v, out_hbm.at[pl.ds(off, CHUNK_A)])
            return carry

        lax.fori_loop(0, nchunk, chunk_body, 0)

    return pos_kernel(posx, posy, posz, edge_index[0], edge_index[1])


def _sc_hist(x, pdist, edge_index, thr, n_nodes, n_edges, k_thr):
    epw = n_edges // NW
    nchunk = epw // CHUNK
    share = n_nodes * WIDTH // NS
    stage = 5000
    nstage = share // stage

    @functools.partial(
        pl.kernel,
        out_type=jax.ShapeDtypeStruct((NC * n_nodes * WIDTH,), jnp.float32),
        mesh=_make_mesh(),
        compiler_params=pltpu.CompilerParams(needs_layout_passes=False),
        scratch_types=[
            pltpu.VMEM((CHUNK,), jnp.int32),
            pltpu.VMEM((CHUNK,), jnp.int32),
            pltpu.VMEM((CHUNK,), jnp.float32),
            pltpu.VMEM((CHUNK, 128), jnp.float32),
            pltpu.VMEM((CHUNK, 128), jnp.float32),
            pltpu.VMEM((128,), jnp.float32),
            pltpu.VMEM((CHUNK,), jnp.int32),
            pltpu.VMEM((CHUNK,), jnp.float32),
            pltpu.VMEM((5008,), jnp.float32),
            pltpu.VMEM_SHARED((n_nodes * WIDTH,), jnp.float32),
            pltpu.SemaphoreType.DMA,
            pltpu.SemaphoreType.DMA,
        ],
    )
    def hist_kernel(x_hbm, pd_hbm, esrc_hbm, edst_hbm, thr_hbm, out_hbm,
                    src_v, dst_v, dp_v, xs_v, xd_v, thr_v,
                    addr_v, ones_v, stage_v, hist_sh, sem0, sem1):
        c = lax.axis_index("c")
        s = lax.axis_index("s")
        wid = s * NC + c

        def zfill(j, carry):
            stage_v[pl.ds(j * 16, 16)] = jnp.zeros((16,), jnp.float32)
            return carry
        lax.fori_loop(0, 5008 // 16, zfill, 0)
        for i in range(nstage):
            pltpu.sync_copy(stage_v.at[pl.ds(0, stage)],
                            hist_sh.at[pl.ds(s * share + i * stage, stage)])
        pltpu.sync_copy(thr_hbm, thr_v)
        for j in range(CHUNK // 16):
            ones_v[pl.ds(j * 16, 16)] = jnp.full((16,), 1.0, jnp.float32)
        plsc.subcore_barrier()

        base = wid * epw

        def chunk_body(i, carry):
            off = base + i * CHUNK
            pltpu.sync_copy(esrc_hbm.at[pl.ds(off, CHUNK)], src_v)
            pltpu.sync_copy(edst_hbm.at[pl.ds(off, CHUNK)], dst_v)
            pltpu.sync_copy(pd_hbm.at[pl.ds(off, CHUNK)], dp_v)
            cp0 = pltpu.async_copy(x_hbm.at[src_v], xs_v, sem0)
            cp1 = pltpu.async_copy(x_hbm.at[dst_v], xd_v, sem1)
            cp0.wait()
            cp1.wait()

            def group_body(g, carry2):
                lidx = g * 16 + jnp.arange(16, dtype=jnp.int32)
                dst16 = dst_v[pl.ds(g * 16, 16)]
                dist2 = dp_v[pl.ds(g * 16, 16)]
                accs = [jnp.zeros((16,), jnp.float32) for _ in range(8)]
                for d in range(128):
                    col = jnp.full((16,), d, jnp.int32)
                    a = plsc.load_gather(xs_v, [lidx, col])
                    b2 = plsc.load_gather(xd_v, [lidx, col])
                    df = a - b2
                    accs[d & 7] = accs[d & 7] + df * df
                dx2 = ((accs[0] + accs[1]) + (accs[2] + accs[3])) + (
                    (accs[4] + accs[5]) + (accs[6] + accs[7]))
                twist = jnp.abs(_nsqrt(dx2) - dist2)
                v = twist * ((k_thr - 1) / 2.0)
                ki = v.astype(jnp.int32)
                k0 = ki + jnp.where(v > ki.astype(jnp.float32), 1, 0)
                k0 = jnp.minimum(k0, k_thr)
                tup = plsc.load_gather(thr_v, [jnp.minimum(k0, k_thr - 1)])
                tdn = plsc.load_gather(thr_v, [jnp.maximum(k0 - 1, 0)])
                inc = jnp.where((k0 <= k_thr - 1) & (twist > tup), 1, 0)
                dec = jnp.where((k0 >= 1) & (twist <= tdn), 1, 0)
                k0 = k0 + inc - dec
                addr = dst16 * WIDTH + k0
                addr_v[pl.ds(g * 16, 16)] = addr
                return carry2

            lax.fori_loop(0, GROUPS, group_body, 0)
            pltpu.sync_copy(ones_v, hist_sh.at[addr_v], add=True)
            return carry

        lax.fori_loop(0, nchunk, chunk_body, 0)
        plsc.subcore_barrier()
        obase = c * (n_nodes * WIDTH) + s * share
        for i in range(nstage):
            pltpu.sync_copy(hist_sh.at[pl.ds(s * share + i * stage, stage)],
                            stage_v.at[pl.ds(0, stage)])
            pltpu.sync_copy(stage_v.at[pl.ds(0, stage)],
                            out_hbm.at[pl.ds(obase + i * stage, stage)])

    return hist_kernel(x, pdist, edge_index[0], edge_index[1], thr)


def _tc_tail_body(hist_ref, wd_ref, bd_ref, msg_ref, conf_ref, k_thr):
    h2 = hist_ref[0] + hist_ref[1]
    h = h2[:, :k_thr]
    extra = h2[:, k_thr:k_thr + 1]
    row = lax.broadcasted_iota(jnp.int32, (k_thr, k_thr), 0)
    colm = lax.broadcasted_iota(jnp.int32, (k_thr, k_thr), 1)
    tri = jnp.where(row <= colm, 1.0, 0.0).astype(jnp.float32)
    s = jnp.dot(h, tri, preferred_element_type=jnp.float32)
    cnt = s[:, k_thr - 1:k_thr] + extra
    mean = s / jnp.maximum(cnt, 1.0)
    msg = jnp.concatenate([mean, s], axis=1)
    msg_ref[...] = msg
    d = jnp.sum(msg * wd_ref[...], axis=1, keepdims=True) + bd_ref[0, 0]
    c0 = 1.0 / (1.0 + jnp.exp(-d))
    c1 = 1.0 / (1.0 + jnp.exp(d))
    conf_ref[...] = jnp.concatenate([c0, c1], axis=1)


def _tc_tail(hist3, wd, bd, n_nodes, k_thr):
    rows = 1000
    grid = (n_nodes // rows,)
    return pl.pallas_call(
        functools.partial(_tc_tail_body, k_thr=k_thr),
        grid=grid,
        in_specs=[
            pl.BlockSpec((NC, rows, WIDTH), lambda i: (0, i, 0)),
            pl.BlockSpec((1, 2 * k_thr), lambda i: (0, 0)),
            pl.BlockSpec((1, 1), lambda i: (0, 0)),
        ],
        out_specs=[
            pl.BlockSpec((rows, 2 * k_thr), lambda i: (i, 0)),
            pl.BlockSpec((rows, 2), lambda i: (i, 0)),
        ],
        out_shape=[
            jax.ShapeDtypeStruct((n_nodes, 2 * k_thr), jnp.float32),
            jax.ShapeDtypeStruct((n_nodes, 2), jnp.float32),
        ],
    )(hist3, wd, bd)


def kernel(x, pos, batch, edge_index, thresholds, W, b):
    n_nodes, _ = x.shape
    n_edges = edge_index.shape[1]
    k_thr = thresholds.shape[1]
    assert n_edges % (NW * CHUNK) == 0
    assert (n_nodes * WIDTH) % (NS * 8) == 0

    thr1 = thresholds.reshape(k_thr)
    posx = pos[:, 0].reshape(n_nodes)
    posy = pos[:, 1].reshape(n_nodes)
    posz = pos[:, 2].reshape(n_nodes)

    pdist = _sc_pos_dist(posx, posy, posz, edge_index, n_nodes, n_edges)
    hist2 = _sc_hist(x, pdist, edge_index, thr1, n_nodes, n_edges, k_thr)
    hist3 = hist2.reshape(NC, n_nodes, WIDTH)

    wd = (W[0] - W[1]).reshape(1, 2 * k_thr)
    bd = (b[0] - b[1]).reshape(1, 1)
    msg, conf = _tc_tail(hist3, wd, bd, n_nodes, k_thr)
    return (msg, conf)


# speedup vs baseline: 3.2975x; 1.1637x over previous
import functools

import jax
import jax.numpy as jnp
from jax import lax
from jax.experimental import pallas as pl
from jax.experimental.pallas import tpu as pltpu
from jax.experimental.pallas import tpu_sc as plsc

NC = 2
NS = 16
NW = NC * NS
CHUNK = 64
GROUPS = CHUNK // 16
EPW = 10240
NCHUNK = EPW // CHUNK
PADE = NW * EPW + 2 * CHUNK
WIDTH = 136
DUMMY_BIN = WIDTH - 1


def _nsqrt(a):
    i = lax.bitcast_convert_type(a, jnp.int32)
    i = 0x5F3759DF - lax.shift_right_logical(i, 1)
    y = lax.bitcast_convert_type(i, jnp.float32)
    for _ in range(4):
        y = y * (1.5 - 0.5 * a * y * y)
    return jnp.where(a > 0.0, a * y, 0.0)


def _make_mesh():
    return plsc.VectorSubcoreMesh(
        core_axis_name="c", subcore_axis_name="s", num_cores=NC, num_subcores=NS
    )


CHUNK_A = 2000


def _sc_pos_dist(posx, posy, posz, edge_index, n_nodes, n_edges):
    epw = n_edges // NW
    nchunk = epw // CHUNK_A

    @functools.partial(
        pl.kernel,
        out_type=jax.ShapeDtypeStruct((n_edges,), jnp.float32),
        mesh=_make_mesh(),
        compiler_params=pltpu.CompilerParams(needs_layout_passes=False),
        scratch_types=[
            pltpu.VMEM((CHUNK_A,), jnp.int32),
            pltpu.VMEM((CHUNK_A,), jnp.int32),
            pltpu.VMEM((CHUNK_A,), jnp.float32),
            pltpu.VMEM((n_nodes,), jnp.float32),
            pltpu.VMEM((n_nodes,), jnp.float32),
            pltpu.VMEM((n_nodes,), jnp.float32),
            pltpu.SemaphoreType.DMA,
        ],
    )
    def pos_kernel(px_hbm, py_hbm, pz_hbm, esrc_hbm, edst_hbm, out_hbm,
                   src_v, dst_v, dpo_v, px_v, py_v, pz_v, sem):
        c = lax.axis_index("c")
        s = lax.axis_index("s")
        wid = s * NC + c
        pltpu.sync_copy(px_hbm, px_v)
        pltpu.sync_copy(py_hbm, py_v)
        pltpu.sync_copy(pz_hbm, pz_v)
        base = wid * epw

        def chunk_body(i, carry):
            off = base + i * CHUNK_A
            cpa = pltpu.async_copy(esrc_hbm.at[pl.ds(off, CHUNK_A)], src_v, sem)
            cpb = pltpu.async_copy(edst_hbm.at[pl.ds(off, CHUNK_A)], dst_v, sem)
            cpa.wait()
            cpb.wait()

            def group_body(g, carry2):
                src16 = src_v[pl.ds(g * 16, 16)]
                dst16 = dst_v[pl.ds(g * 16, 16)]
                dp2 = jnp.zeros((16,), jnp.float32)
                for pv in (px_v, py_v, pz_v):
                    a = plsc.load_gather(pv, [dst16])
                    b2 = plsc.load_gather(pv, [src16])
                    df = a - b2
                    dp2 = dp2 + df * df
                dpo_v[pl.ds(g * 16, 16)] = _nsqrt(dp2)
                return carry2

            lax.fori_loop(0, CHUNK_A // 16, group_body, 0)
            pltpu.sync_copy(dpo_v, out_hbm.at[pl.ds(off, CHUNK_A)])
            return carry

        lax.fori_loop(0, nchunk, chunk_body, 0)

    return pos_kernel(posx, posy, posz, edge_index[0], edge_index[1])


def _sc_hist(x, pdist, esrc, edst, thr, n_nodes, n_edges, k_thr):
    share = n_nodes * WIDTH // NS
    stage = 5000
    nstage = share // stage

    @functools.partial(
        pl.kernel,
        out_type=jax.ShapeDtypeStruct((NC * n_nodes * WIDTH,), jnp.float32),
        mesh=_make_mesh(),
        compiler_params=pltpu.CompilerParams(needs_layout_passes=False),
        scratch_types=[
            pltpu.VMEM((CHUNK,), jnp.int32),
            pltpu.VMEM((CHUNK,), jnp.int32),
            pltpu.VMEM((CHUNK,), jnp.int32),
            pltpu.VMEM((CHUNK,), jnp.int32),
            pltpu.VMEM((CHUNK,), jnp.float32),
            pltpu.VMEM((CHUNK,), jnp.float32),
            pltpu.VMEM((CHUNK, 128), jnp.float32),
            pltpu.VMEM((CHUNK, 128), jnp.float32),
            pltpu.VMEM((CHUNK, 128), jnp.float32),
            pltpu.VMEM((CHUNK, 128), jnp.float32),
            pltpu.VMEM((128,), jnp.float32),
            pltpu.VMEM((CHUNK,), jnp.int32),
            pltpu.VMEM((CHUNK,), jnp.int32),
            pltpu.VMEM((CHUNK,), jnp.float32),
            pltpu.VMEM((5008,), jnp.float32),
            pltpu.VMEM_SHARED((n_nodes * WIDTH,), jnp.float32),
            pltpu.SemaphoreType.DMA,
            pltpu.SemaphoreType.DMA,
            pltpu.SemaphoreType.DMA,
            pltpu.SemaphoreType.DMA,
            pltpu.SemaphoreType.DMA,
            pltpu.SemaphoreType.DMA,
        ],
    )
    def hist_kernel(x_hbm, pd_hbm, esrc_hbm, edst_hbm, thr_hbm, out_hbm,
                    src0, src1, dst0, dst1, dp0, dp1,
                    xs0, xs1, xd0, xd1, thr_v, addr0, addr1,
                    ones_v, stage_v, hist_sh,
                    esem0, esem1, gsem0, gsem1, ssem0, ssem1):
        srcs, dsts, dps = [src0, src1], [dst0, dst1], [dp0, dp1]
        xss, xds, addrs = [xs0, xs1], [xd0, xd1], [addr0, addr1]
        esems, gsems, ssems = [esem0, esem1], [gsem0, gsem1], [ssem0, ssem1]
        c = lax.axis_index("c")
        s = lax.axis_index("s")
        wid = s * NC + c

        def zfill(j, carry):
            stage_v[pl.ds(j * 16, 16)] = jnp.zeros((16,), jnp.float32)
            return carry
        lax.fori_loop(0, 5008 // 16, zfill, 0)
        for i in range(nstage):
            pltpu.sync_copy(stage_v.at[pl.ds(0, stage)],
                            hist_sh.at[pl.ds(s * share + i * stage, stage)])
        pltpu.sync_copy(thr_hbm, thr_v)
        for j in range(CHUNK // 16):
            ones_v[pl.ds(j * 16, 16)] = jnp.full((16,), 1.0, jnp.float32)
            addr0[pl.ds(j * 16, 16)] = jnp.full((16,), DUMMY_BIN, jnp.int32)
            addr1[pl.ds(j * 16, 16)] = jnp.full((16,), DUMMY_BIN, jnp.int32)
        plsc.subcore_barrier()

        base = wid * EPW

        def edge_issue(i, p):
            off = base + i * CHUNK
            pltpu.async_copy(esrc_hbm.at[pl.ds(off, CHUNK)], srcs[p], esems[p])
            pltpu.async_copy(edst_hbm.at[pl.ds(off, CHUNK)], dsts[p], esems[p])
            pltpu.async_copy(pd_hbm.at[pl.ds(off, CHUNK)], dps[p], esems[p])

        def edge_wait(p):
            pltpu.make_async_copy(
                esrc_hbm.at[pl.ds(0, CHUNK)], srcs[p], esems[p]).wait()
            pltpu.make_async_copy(
                edst_hbm.at[pl.ds(0, CHUNK)], dsts[p], esems[p]).wait()
            pltpu.make_async_copy(
                pd_hbm.at[pl.ds(0, CHUNK)], dps[p], esems[p]).wait()

        def gather_issue(p):
            pltpu.async_copy(x_hbm.at[srcs[p]], xss[p], gsems[p])
            pltpu.async_copy(x_hbm.at[dsts[p]], xds[p], gsems[p])

        def gather_wait(p):
            pltpu.make_async_copy(x_hbm.at[srcs[p]], xss[p], gsems[p]).wait()
            pltpu.make_async_copy(x_hbm.at[dsts[p]], xds[p], gsems[p]).wait()

        def scatter_issue(p):
            pltpu.async_copy(ones_v, hist_sh.at[addrs[p]], ssems[p], add=True)

        def scatter_wait(p):
            pltpu.make_async_copy(ones_v, hist_sh.at[addrs[p]], ssems[p]).wait()

        def compute(i, p):
            off = base + i * CHUNK
            xs_v, xd_v = xss[p], xds[p]
            dst_v, dp_v, addr_v = dsts[p], dps[p], addrs[p]

            def group_body(g, carry2):
                lidx = g * 16 + jnp.arange(16, dtype=jnp.int32)
                dst16 = dst_v[pl.ds(g * 16, 16)]
                dist2 = dp_v[pl.ds(g * 16, 16)]
                accs = [jnp.zeros((16,), jnp.float32) for _ in range(8)]
                for d in range(128):
                    col = jnp.full((16,), d, jnp.int32)
                    a = plsc.load_gather(xs_v, [lidx, col])
                    b2 = plsc.load_gather(xd_v, [lidx, col])
                    df = a - b2
                    accs[d & 7] = accs[d & 7] + df * df
                dx2 = ((accs[0] + accs[1]) + (accs[2] + accs[3])) + (
                    (accs[4] + accs[5]) + (accs[6] + accs[7]))
                twist = jnp.abs(_nsqrt(dx2) - dist2)
                v = twist * ((k_thr - 1) / 2.0)
                ki = v.astype(jnp.int32)
                k0 = ki + jnp.where(v > ki.astype(jnp.float32), 1, 0)
                k0 = jnp.minimum(k0, k_thr)
                tup = plsc.load_gather(thr_v, [jnp.minimum(k0, k_thr - 1)])
                tdn = plsc.load_gather(thr_v, [jnp.maximum(k0 - 1, 0)])
                inc = jnp.where((k0 <= k_thr - 1) & (twist > tup), 1, 0)
                dec = jnp.where((k0 >= 1) & (twist <= tdn), 1, 0)
                k0 = k0 + inc - dec
                addr = dst16 * WIDTH + k0
                gvec = off + lidx
                addr = jnp.where(gvec >= n_edges, DUMMY_BIN, addr)
                addr_v[pl.ds(g * 16, 16)] = addr
                return carry2

            lax.fori_loop(0, GROUPS, group_body, 0)

        scatter_issue(0)
        scatter_issue(1)
        edge_issue(0, 0)
        edge_issue(1, 1)
        edge_wait(0)
        gather_issue(0)

        def pair_body(j, carry):
            for q in range(2):
                i = 2 * j + q
                p = q
                edge_wait(1 - p)
                gather_issue(1 - p)
                gather_wait(p)
                scatter_wait(p)
                compute(i, p)
                scatter_issue(p)
                edge_issue(i + 2, p)
            return carry

        lax.fori_loop(0, NCHUNK // 2, pair_body, 0)
        scatter_wait(0)
        scatter_wait(1)
        gather_wait(0)
        edge_wait(1)
        plsc.subcore_barrier()

        obase = c * (n_nodes * WIDTH) + s * share
        for i in range(nstage):
            pltpu.sync_copy(hist_sh.at[pl.ds(s * share + i * stage, stage)],
                            stage_v.at[pl.ds(0, stage)])
            pltpu.sync_copy(stage_v.at[pl.ds(0, stage)],
                            out_hbm.at[pl.ds(obase + i * stage, stage)])

    return hist_kernel(x, pdist, esrc, edst, thr)


def _tc_tail_body(hist_ref, wd_ref, bd_ref, msg_ref, conf_ref, k_thr):
    h2 = hist_ref[0] + hist_ref[1]
    h = h2[:, :k_thr]
    extra = h2[:, k_thr:k_thr + 1]
    row = lax.broadcasted_iota(jnp.int32, (k_thr, k_thr), 0)
    colm = lax.broadcasted_iota(jnp.int32, (k_thr, k_thr), 1)
    tri = jnp.where(row <= colm, 1.0, 0.0).astype(jnp.float32)
    s = jnp.dot(h, tri, preferred_element_type=jnp.float32)
    cnt = s[:, k_thr - 1:k_thr] + extra
    mean = s / jnp.maximum(cnt, 1.0)
    msg = jnp.concatenate([mean, s], axis=1)
    msg_ref[...] = msg
    d = jnp.sum(msg * wd_ref[...], axis=1, keepdims=True) + bd_ref[0, 0]
    c0 = 1.0 / (1.0 + jnp.exp(-d))
    c1 = 1.0 / (1.0 + jnp.exp(d))
    conf_ref[...] = jnp.concatenate([c0, c1], axis=1)


def _tc_tail(hist3, wd, bd, n_nodes, k_thr):
    rows = 1000
    grid = (n_nodes // rows,)
    return pl.pallas_call(
        functools.partial(_tc_tail_body, k_thr=k_thr),
        grid=grid,
        in_specs=[
            pl.BlockSpec((NC, rows, WIDTH), lambda i: (0, i, 0)),
            pl.BlockSpec((1, 2 * k_thr), lambda i: (0, 0)),
            pl.BlockSpec((1, 1), lambda i: (0, 0)),
        ],
        out_specs=[
            pl.BlockSpec((rows, 2 * k_thr), lambda i: (i, 0)),
            pl.BlockSpec((rows, 2), lambda i: (i, 0)),
        ],
        out_shape=[
            jax.ShapeDtypeStruct((n_nodes, 2 * k_thr), jnp.float32),
            jax.ShapeDtypeStruct((n_nodes, 2), jnp.float32),
        ],
    )(hist3, wd, bd)


def kernel(x, pos, batch, edge_index, thresholds, W, b):
    n_nodes, _ = x.shape
    n_edges = edge_index.shape[1]
    k_thr = thresholds.shape[1]
    assert n_edges % (NW * CHUNK_A) == 0
    assert NW * EPW >= n_edges
    assert (n_nodes * WIDTH) % (NS * 8) == 0

    thr1 = thresholds.reshape(k_thr)
    posx = pos[:, 0].reshape(n_nodes)
    posy = pos[:, 1].reshape(n_nodes)
    posz = pos[:, 2].reshape(n_nodes)

    pdist = _sc_pos_dist(posx, posy, posz, edge_index, n_nodes, n_edges)
    padn = PADE - n_edges
    esrc_p = jnp.pad(edge_index[0], (0, padn))
    edst_p = jnp.pad(edge_index[1], (0, padn))
    pdist_p = jnp.pad(pdist, (0, padn))
    hist2 = _sc_hist(x, pdist_p, esrc_p, edst_p, thr1,
                     n_nodes, n_edges, k_thr)
    hist3 = hist2.reshape(NC, n_nodes, WIDTH)

    wd = (W[0] - W[1]).reshape(1, 2 * k_thr)
    bd = (b[0] - b[1]).reshape(1, 1)
    msg, conf = _tc_tail(hist3, wd, bd, n_nodes, k_thr)
    return (msg, conf)
